# baseline (speedup 1.0000x reference)
import functools

import jax
import jax.numpy as jnp
from jax.experimental import pallas as pl

NU = 100000; NI = 100000; H = 128; OUT = 128
N_I0 = 200000; N_I1 = 50000; N_I2 = 16384; N_U1 = 8192; N_U2 = 4096

BLK = 512
_P = jax.lax.Precision.HIGHEST


def _pad_rows(x, n):
    return jnp.pad(x, ((0, n - x.shape[0]), (0, 0)))


def _sage2_body(x_ref, s_ref, d_ref, wsa_ref, wna_ref, ba_ref,
                wsb_ref, wnb_ref, bb_ref, oa_ref, ob_ref):
    deg = jnp.maximum(d_ref[:, 0:1], 1.0)
    mean = s_ref[...] * (1.0 / deg)
    x = x_ref[...]
    oa_ref[...] = jnp.maximum(
        jnp.dot(x, wsa_ref[...], precision=_P)
        + jnp.dot(mean, wna_ref[...], precision=_P) + ba_ref[...], 0.0)
    ob_ref[...] = jnp.maximum(
        jnp.dot(x, wsb_ref[...], precision=_P)
        + jnp.dot(mean, wnb_ref[...], precision=_P) + bb_ref[...], 0.0)


def _sage_dual(x, sums, deg, Wsa, Wna, ba, Wsb, Wnb, bb):
    n = x.shape[0]
    grid = (n // BLK,)
    row = pl.BlockSpec((BLK, H), lambda i: (i, 0))
    w = pl.BlockSpec((H, H), lambda i: (0, 0))
    bsp = pl.BlockSpec((1, H), lambda i: (0, 0))
    dsp = pl.BlockSpec((BLK, 1), lambda i: (i, 0))
    return pl.pallas_call(
        _sage2_body,
        grid=grid,
        in_specs=[row, row, dsp, w, w, bsp, w, w, bsp],
        out_specs=[row, row],
        out_shape=[jax.ShapeDtypeStruct((n, H), jnp.float32)] * 2,
    )(x, sums, deg[:, None], Wsa, Wna, ba[None, :], Wsb, Wnb, bb[None, :])


def _sage1_body(x_ref, s_ref, d_ref, ws_ref, wn_ref, b_ref, o_ref):
    deg = jnp.maximum(d_ref[:, 0:1], 1.0)
    mean = s_ref[...] * (1.0 / deg)
    o_ref[...] = jnp.maximum(
        jnp.dot(x_ref[...], ws_ref[...], precision=_P)
        + jnp.dot(mean, wn_ref[...], precision=_P) + b_ref[...], 0.0)


def _sage_one(x, sums, deg, Ws, Wn, b):
    n = x.shape[0]
    grid = (n // BLK,)
    row = pl.BlockSpec((BLK, H), lambda i: (i, 0))
    w = pl.BlockSpec((H, H), lambda i: (0, 0))
    bsp = pl.BlockSpec((1, H), lambda i: (0, 0))
    dsp = pl.BlockSpec((BLK, 1), lambda i: (i, 0))
    return pl.pallas_call(
        _sage1_body,
        grid=grid,
        in_specs=[row, row, dsp, w, w, bsp],
        out_specs=row,
        out_shape=jax.ShapeDtypeStruct((n, H), jnp.float32),
    )(x, sums, deg[:, None], Ws, Wn, b[None, :])


def _sage_lin_body(x_ref, s_ref, d_ref, ws_ref, wn_ref, b_ref,
                   lw_ref, lb_ref, o_ref):
    deg = jnp.maximum(d_ref[:, 0:1], 1.0)
    mean = s_ref[...] * (1.0 / deg)
    u = jnp.maximum(
        jnp.dot(x_ref[...], ws_ref[...], precision=_P)
        + jnp.dot(mean, wn_ref[...], precision=_P) + b_ref[...], 0.0)
    o_ref[...] = jnp.dot(u, lw_ref[...], precision=_P) + lb_ref[...]


def _sage_lin(x, sums, deg, Ws, Wn, b, lW, lb):
    n = x.shape[0]
    grid = (n // BLK,)
    row = pl.BlockSpec((BLK, H), lambda i: (i, 0))
    w = pl.BlockSpec((H, H), lambda i: (0, 0))
    bsp = pl.BlockSpec((1, H), lambda i: (0, 0))
    dsp = pl.BlockSpec((BLK, 1), lambda i: (i, 0))
    return pl.pallas_call(
        _sage_lin_body,
        grid=grid,
        in_specs=[row, row, dsp, w, w, bsp, w, bsp],
        out_specs=row,
        out_shape=jax.ShapeDtypeStruct((n, H), jnp.float32),
    )(x, sums, deg[:, None], Ws, Wn, b[None, :], lW, lb[None, :])


def _dec_body(zs_ref, zd_ref, w1a_ref, w1b_ref, b1_ref, w2_ref, b2_ref, o_ref):
    t = jnp.maximum(
        jnp.dot(zs_ref[...], w1a_ref[...], precision=_P)
        + jnp.dot(zd_ref[...], w1b_ref[...], precision=_P) + b1_ref[...], 0.0)
    o_ref[...] = jnp.sum(t * w2_ref[...], axis=1, keepdims=True) + b2_ref[0, 0]


def _decoder(zs, zd, W1a, W1b, b1, w2row, b2):
    n = zs.shape[0]
    grid = (n // BLK,)
    row = pl.BlockSpec((BLK, H), lambda i: (i, 0))
    w = pl.BlockSpec((H, H), lambda i: (0, 0))
    bsp = pl.BlockSpec((1, H), lambda i: (0, 0))
    osp = pl.BlockSpec((BLK, 1), lambda i: (i, 0))
    return pl.pallas_call(
        _dec_body,
        grid=grid,
        in_specs=[row, row, w, w, bsp, bsp, pl.BlockSpec((1, 1), lambda i: (0, 0))],
        out_specs=osp,
        out_shape=jax.ShapeDtypeStruct((n, 1), jnp.float32),
    )(zs, zd, W1a, W1b, b1[None, :], w2row, b2[None, :])


def _take(tab, idx):
    return tab.at[idx].get(mode="promise_in_bounds")


def _seg_sum_deg(msg, dst, nseg):
    s = jax.ops.segment_sum(msg, dst, num_segments=nseg)
    d = jax.ops.segment_sum(jnp.ones((dst.shape[0],), jnp.float32), dst,
                            num_segments=nseg)
    return s, d


def kernel(item_ids, user_ids, ii0_src, ii0_dst, ii1_src, ii1_dst, iu0_src, iu0_dst, iu1_src, iu1_dst, pos_src, pos_dst, neg_src, neg_dst, item_emb_w, user_emb_w, ie1_Ws, ie1_Wn, ie1_b, ie2_Ws, ie2_Wn, ie2_b, ue1_Ws, ue1_Wn, ue1_b, ue2_Ws, ue2_Wn, ue2_b, ue3_Ws, ue3_Wn, ue3_b, lin_W, lin_b, dec1_W, dec1_b, dec2_W, dec2_b):
    NP1 = 50176

    idx50 = item_ids[:N_I1]
    x0 = _pad_rows(_take(item_emb_w, idx50), NP1)
    msg0 = _take(item_emb_w, _take(item_ids, ii0_src))
    sum0, deg0 = _seg_sum_deg(msg0, ii0_dst, NP1)

    h, item_x = _sage_dual(x0, sum0, deg0, ie1_Ws, ie1_Wn, ie1_b,
                           ue1_Ws, ue1_Wn, ue1_b)

    sum1, deg1 = _seg_sum_deg(_take(h, ii1_src), ii1_dst, N_I2)
    z_item = _sage_one(h[:N_I2], sum1, deg1, ie2_Ws, ie2_Wn, ie2_b)

    xu = _take(user_emb_w, user_ids)
    msgU0 = _take(item_emb_w, _take(item_ids, iu0_src))
    sumU0, degU0 = _seg_sum_deg(msgU0, iu0_dst, N_U1)
    user_x = _sage_one(xu, sumU0, degU0, ue2_Ws, ue2_Wn, ue2_b)

    sumU1, degU1 = _seg_sum_deg(_take(item_x, iu1_src), iu1_dst, N_U2)
    z_user = _sage_lin(user_x[:N_U2], sumU1, degU1, ue3_Ws, ue3_Wn, ue3_b,
                       lin_W, lin_b)

    zs = jnp.concatenate([_take(z_user, pos_src), _take(z_user, neg_src)], axis=0)
    zd = jnp.concatenate([_take(z_item, pos_dst), _take(z_item, neg_dst)], axis=0)
    z = _decoder(zs, zd, dec1_W[:H], dec1_W[H:], dec1_b, dec2_W[:, 0][None, :],
                 dec2_b)
    return z.reshape(-1)


# speedup vs baseline: 2.5858x; 2.5858x over previous
import functools

import jax
import jax.numpy as jnp
from jax import lax
from jax.experimental import pallas as pl
from jax.experimental.pallas import tpu as pltpu
from jax.experimental.pallas import tpu_sc as plsc

NU = 100000; NI = 100000; H = 128; OUT = 128
N_I0 = 200000; N_I1 = 50000; N_I2 = 16384; N_U1 = 8192; N_U2 = 4096

BLK = 512
NC = 2
NS = 16
NW = NC * NS
K = 128
BE = 1024
_P = jax.lax.Precision.HIGHEST
_SENT = 1 << 29

_mesh = plsc.VectorSubcoreMesh(core_axis_name="c", subcore_axis_name="s")


def _sc_gather(table, idx):
    B = idx.shape[0]
    V, Hh = table.shape
    b_per_w = B // NW
    kg = next(k for k in (256, 224, 160, 128, 96, 64, 32, 16, 8)
              if b_per_w % k == 0)
    nb = b_per_w // kg

    @functools.partial(
        pl.kernel,
        out_type=jax.ShapeDtypeStruct((B, Hh), jnp.float32),
        mesh=_mesh,
        scratch_types=[
            pltpu.VMEM((b_per_w,), jnp.int32),
            pltpu.VMEM((kg, Hh), jnp.float32),
            pltpu.SemaphoreType.DMA,
        ],
    )
    def k(table_h, idx_h, out_h, idxv, rows, sem):
        wid = lax.axis_index("s") * NC + lax.axis_index("c")
        base = wid * b_per_w
        pltpu.sync_copy(idx_h.at[pl.ds(base, b_per_w)], idxv)
        for b in range(nb):
            pltpu.async_copy(table_h.at[idxv.at[pl.ds(b * kg, kg)]], rows,
                             sem).wait()
            pltpu.sync_copy(rows, out_h.at[pl.ds(base + b * kg, kg)])

    return k(table, idx)


def _sc_deg(dst, nd):
    EP = dst.shape[0]
    SLAB = EP // NW
    NSUB = SLAB // 512
    zrow = jnp.zeros((nd,), jnp.float32)

    def body(dst_h, z_h, out_h, hist, dstv, sem):
        c = lax.axis_index("c")
        s = lax.axis_index("s")
        wid = s * NC + c
        slab0 = wid * SLAB
        onev = jnp.zeros((16,), jnp.float32) + 1.0
        pltpu.sync_copy(z_h, hist)

        def sub(j, carry):
            pltpu.sync_copy(dst_h.at[pl.ds(slab0 + j * 512, 512)], dstv)

            def inner(i, carry2):
                d = dstv[pl.ds(i * 16, 16)]
                m = d < nd
                plsc.addupdate_scatter(hist, [d], onev, mask=m)
                return carry2

            return lax.fori_loop(0, 512 // 16, inner, carry)

        lax.fori_loop(0, NSUB, sub, jnp.int32(0))
        pltpu.sync_copy(hist, out_h.at[wid])

    f = functools.partial(
        pl.kernel,
        out_type=jax.ShapeDtypeStruct((NW, nd), jnp.float32),
        mesh=_mesh,
        scratch_types=[
            pltpu.VMEM((nd,), jnp.float32),
            pltpu.VMEM((512,), jnp.int32),
            pltpu.SemaphoreType.DMA,
        ],
        compiler_params=pltpu.CompilerParams(needs_layout_passes=False),
    )(body)
    return f(dst, zrow)


def _pad_edges(src, dst, bec):
    e = src.shape[0]
    ep = -(-e // (NS * bec)) * (NS * bec)
    if ep == e:
        return src, dst
    return (jnp.pad(src, (0, ep - e)),
            jnp.pad(dst, (0, ep - e), constant_values=_SENT))


def _sage2_body(x_ref, s_ref, d_ref, wsa_ref, wna_ref, ba_ref,
                wsb_ref, wnb_ref, bb_ref, oa_ref, ob_ref):
    deg = jnp.maximum(d_ref[:, 0:1], 1.0)
    mean = s_ref[...] * (1.0 / deg)
    x = x_ref[...]
    oa_ref[...] = jnp.maximum(
        jnp.dot(x, wsa_ref[...], precision=_P)
        + jnp.dot(mean, wna_ref[...], precision=_P) + ba_ref[...], 0.0)
    ob_ref[...] = jnp.maximum(
        jnp.dot(x, wsb_ref[...], precision=_P)
        + jnp.dot(mean, wnb_ref[...], precision=_P) + bb_ref[...], 0.0)


def _sage_dual(x, sums, deg, Wsa, Wna, ba, Wsb, Wnb, bb):
    n = x.shape[0]
    grid = (n // BLK,)
    row = pl.BlockSpec((BLK, H), lambda i: (i, 0))
    w = pl.BlockSpec((H, H), lambda i: (0, 0))
    bsp = pl.BlockSpec((1, H), lambda i: (0, 0))
    dsp = pl.BlockSpec((BLK, 1), lambda i: (i, 0))
    return pl.pallas_call(
        _sage2_body,
        grid=grid,
        in_specs=[row, row, dsp, w, w, bsp, w, w, bsp],
        out_specs=[row, row],
        out_shape=[jax.ShapeDtypeStruct((n, H), jnp.float32)] * 2,
    )(x, sums, deg[:, None], Wsa, Wna, ba[None, :], Wsb, Wnb, bb[None, :])


def _sage1_body(x_ref, s_ref, d_ref, ws_ref, wn_ref, b_ref, o_ref):
    deg = jnp.maximum(d_ref[:, 0:1], 1.0)
    mean = s_ref[...] * (1.0 / deg)
    o_ref[...] = jnp.maximum(
        jnp.dot(x_ref[...], ws_ref[...], precision=_P)
        + jnp.dot(mean, wn_ref[...], precision=_P) + b_ref[...], 0.0)


def _sage_one(x, sums, deg, Ws, Wn, b):
    n = x.shape[0]
    grid = (n // BLK,)
    row = pl.BlockSpec((BLK, H), lambda i: (i, 0))
    w = pl.BlockSpec((H, H), lambda i: (0, 0))
    bsp = pl.BlockSpec((1, H), lambda i: (0, 0))
    dsp = pl.BlockSpec((BLK, 1), lambda i: (i, 0))
    return pl.pallas_call(
        _sage1_body,
        grid=grid,
        in_specs=[row, row, dsp, w, w, bsp],
        out_specs=row,
        out_shape=jax.ShapeDtypeStruct((n, H), jnp.float32),
    )(x, sums, deg[:, None], Ws, Wn, b[None, :])


def _sage_lin_body(x_ref, s_ref, d_ref, ws_ref, wn_ref, b_ref,
                   lw_ref, lb_ref, o_ref):
    deg = jnp.maximum(d_ref[:, 0:1], 1.0)
    mean = s_ref[...] * (1.0 / deg)
    u = jnp.maximum(
        jnp.dot(x_ref[...], ws_ref[...], precision=_P)
        + jnp.dot(mean, wn_ref[...], precision=_P) + b_ref[...], 0.0)
    o_ref[...] = jnp.dot(u, lw_ref[...], precision=_P) + lb_ref[...]


def _sage_lin(x, sums, deg, Ws, Wn, b, lW, lb):
    n = x.shape[0]
    grid = (n // BLK,)
    row = pl.BlockSpec((BLK, H), lambda i: (i, 0))
    w = pl.BlockSpec((H, H), lambda i: (0, 0))
    bsp = pl.BlockSpec((1, H), lambda i: (0, 0))
    dsp = pl.BlockSpec((BLK, 1), lambda i: (i, 0))
    return pl.pallas_call(
        _sage_lin_body,
        grid=grid,
        in_specs=[row, row, dsp, w, w, bsp, w, bsp],
        out_specs=row,
        out_shape=jax.ShapeDtypeStruct((n, H), jnp.float32),
    )(x, sums, deg[:, None], Ws, Wn, b[None, :], lW, lb[None, :])


def _dec_body(zs_ref, zd_ref, w1a_ref, w1b_ref, b1_ref, w2_ref, b2_ref, o_ref):
    t = jnp.maximum(
        jnp.dot(zs_ref[...], w1a_ref[...], precision=_P)
        + jnp.dot(zd_ref[...], w1b_ref[...], precision=_P) + b1_ref[...], 0.0)
    o_ref[...] = jnp.sum(t * w2_ref[...], axis=1, keepdims=True) + b2_ref[0, 0]


def _decoder(zs, zd, W1a, W1b, b1, w2row, b2):
    n = zs.shape[0]
    grid = (n // BLK,)
    row = pl.BlockSpec((BLK, H), lambda i: (i, 0))
    w = pl.BlockSpec((H, H), lambda i: (0, 0))
    bsp = pl.BlockSpec((1, H), lambda i: (0, 0))
    osp = pl.BlockSpec((BLK, 1), lambda i: (i, 0))
    return pl.pallas_call(
        _dec_body,
        grid=grid,
        in_specs=[row, row, w, w, bsp, bsp, pl.BlockSpec((1, 1), lambda i: (0, 0))],
        out_specs=osp,
        out_shape=jax.ShapeDtypeStruct((n, 1), jnp.float32),
    )(zs, zd, W1a, W1b, b1[None, :], w2row, b2[None, :])


def _pad_ids(idx, n):
    return jnp.pad(idx, (0, n - idx.shape[0]))


def _seg_sum(msg, dst, nseg):
    return jax.ops.segment_sum(msg, dst, num_segments=nseg)


def kernel(item_ids, user_ids, ii0_src, ii0_dst, ii1_src, ii1_dst, iu0_src, iu0_dst, iu1_src, iu1_dst, pos_src, pos_dst, neg_src, neg_dst, item_emb_w, user_emb_w, ie1_Ws, ie1_Wn, ie1_b, ie2_Ws, ie2_Wn, ie2_b, ue1_Ws, ue1_Wn, ue1_b, ue2_Ws, ue2_Wn, ue2_b, ue3_Ws, ue3_Wn, ue3_b, lin_W, lin_b, dec1_W, dec1_b, dec2_W, dec2_b):
    NP1 = 50176

    x_item = _sc_gather(item_emb_w, _pad_ids(item_ids, 204800))
    x0 = _sc_gather(item_emb_w, _pad_ids(item_ids[:N_I1], NP1))
    xu = _sc_gather(user_emb_w, user_ids)

    _, d0 = _pad_edges(ii0_src, ii0_dst, 512)
    deg0 = jnp.sum(_sc_deg(d0, NP1), axis=0)
    deg1 = jnp.sum(_sc_deg(ii1_dst, N_I2), axis=0)
    degU0 = jnp.sum(_sc_deg(iu0_dst, N_U1), axis=0)
    degU1 = jnp.sum(_sc_deg(iu1_dst, N_U2), axis=0)

    msg0 = x_item.at[ii0_src].get(mode="promise_in_bounds")
    sum0 = _seg_sum(msg0, ii0_dst, NP1)

    h, item_x = _sage_dual(x0, sum0, deg0, ie1_Ws, ie1_Wn, ie1_b,
                           ue1_Ws, ue1_Wn, ue1_b)

    sum1 = _seg_sum(h.at[ii1_src].get(mode="promise_in_bounds"), ii1_dst,
                    N_I2)
    z_item = _sage_one(h[:N_I2], sum1, deg1, ie2_Ws, ie2_Wn, ie2_b)

    sumU0 = _seg_sum(x_item.at[iu0_src].get(mode="promise_in_bounds"),
                     iu0_dst, N_U1)
    user_x = _sage_one(xu, sumU0, degU0, ue2_Ws, ue2_Wn, ue2_b)

    sumU1 = _seg_sum(item_x.at[iu1_src].get(mode="promise_in_bounds"),
                     iu1_dst, N_U2)
    z_user = _sage_lin(user_x[:N_U2], sumU1, degU1, ue3_Ws, ue3_Wn, ue3_b,
                       lin_W, lin_b)

    zs = _sc_gather(z_user, jnp.concatenate([pos_src, neg_src]))
    zd = _sc_gather(z_item, jnp.concatenate([pos_dst, neg_dst]))
    z = _decoder(zs, zd, dec1_W[:H], dec1_W[H:], dec1_b, dec2_W[:, 0][None, :],
                 dec2_b)
    return z.reshape(-1)
